# baseline (speedup 1.0000x reference)
import functools

import jax
import jax.numpy as jnp
from jax import lax
from jax.experimental import pallas as pl
from jax.experimental.pallas import tpu as pltpu
from jax.experimental.pallas import tpu_sc as plsc

N_NODES = 10000
N_PAD = 10240
D = 128
NC, NS = 2, 16
NW = NC * NS
B = 128
NB = 80
NBT = NW * NB
NB0, NB1 = 40, 120
NBM = max(NB0, NB1)
E_PAD = NBT * B
STRIPE = N_PAD // NS
ROW_BLK = 128
ZB = 8


def _sc_mesh():
    return plsc.VectorSubcoreMesh(
        core_axis_name="c", subcore_axis_name="s",
        num_cores=NC, num_subcores=NS)


HB = 128


def _sc_degree(dst_t):
    @functools.partial(
        pl.kernel,
        out_type=jax.ShapeDtypeStruct((NC, HB, 128), jnp.float32),
        mesh=_sc_mesh(),
        compiler_params=pltpu.CompilerParams(needs_layout_passes=False),
        scratch_types=[
            pltpu.VMEM((NB, B), jnp.int32),
            pltpu.VMEM((HB, 128), jnp.float32),
            pltpu.VMEM((1, 128), jnp.int32),
            pltpu.VMEM_SHARED((HB, 128), jnp.float32),
        ],
    )
    def k(dst_hbm, out_hbm, dst_v, hist_v, iden_v, acc_sh):
        c = lax.axis_index("c")
        s = lax.axis_index("s")
        wid = c * NS + s

        def zero_row(i, _):
            def zero_lane(jj, _):
                hist_v[i, pl.ds(jj * 16, 16)] = jnp.zeros((16,), jnp.float32)
                return 0
            return lax.fori_loop(0, 128 // 16, zero_lane, 0)
        lax.fori_loop(0, HB, zero_row, 0)
        for kk in range(128 // 16):
            iden_v[0, pl.ds(kk * 16, 16)] = lax.iota(jnp.int32, 16) + 16 * kk
        pltpu.sync_copy(hist_v.at[pl.ds(s * (HB // NS), HB // NS)],
                        acc_sh.at[pl.ds(s * (HB // NS), HB // NS)])
        plsc.subcore_barrier()

        pltpu.sync_copy(dst_hbm.at[wid], dst_v)
        ones16 = jnp.ones((16,), jnp.float32)
        def hrow(jj, _):
            def hlane(kk, _):
                v = dst_v[jj, pl.ds(kk * 16, 16)]
                plsc.addupdate_scatter(
                    hist_v, [jnp.right_shift(v, 7), jnp.bitwise_and(v, 127)],
                    ones16)
                return 0
            return lax.fori_loop(0, B // 16, hlane, 0)
        lax.fori_loop(0, NB, hrow, 0)

        pltpu.sync_copy(hist_v, acc_sh.at[iden_v.at[0]], add=True)
        plsc.subcore_barrier()

        base = s * (HB // NS)
        pltpu.sync_copy(acc_sh.at[pl.ds(base, HB // NS)],
                        out_hbm.at[c].at[pl.ds(base, HB // NS)])
    return k(dst_t)


def _sc_scatter(y_pad, src_f, dst_f):
    @functools.partial(
        pl.kernel,
        out_type=jax.ShapeDtypeStruct((NC, N_PAD, D), jnp.float32),
        mesh=_sc_mesh(),
        scratch_types=[
            pltpu.VMEM((NBM, B), jnp.int32),
            pltpu.VMEM((NBM, B), jnp.int32),
            pltpu.VMEM((B, D), jnp.float32),
            pltpu.VMEM((ZB, D), jnp.float32),
            pltpu.VMEM_SHARED((N_PAD, D), jnp.float32),
            pltpu.SemaphoreType.DMA,
        ],
    )
    def k(y_hbm, src_hbm, dst_hbm, out_hbm,
          src_v, dst_v, rows_v, zero_v, acc_sh, sem):
        c = lax.axis_index("c")
        s = lax.axis_index("s")

        def zero_row(i, _):
            def zero_lane(jj, _):
                zero_v[i, pl.ds(jj * 16, 16)] = jnp.zeros((16,), jnp.float32)
                return 0
            return lax.fori_loop(0, D // 16, zero_lane, 0)
        lax.fori_loop(0, ZB, zero_row, 0)

        base = s * STRIPE
        def zero_stripe(kk, _):
            pltpu.sync_copy(zero_v, acc_sh.at[pl.ds(base + kk * ZB, ZB)])
            return 0
        lax.fori_loop(0, STRIPE // ZB, zero_stripe, 0)
        plsc.subcore_barrier()

        batch0 = jnp.where(c == 0, s * NB0, NS * NB0 + s * NB1)
        cnt = jnp.where(c == 0, NB0, NB1)
        pltpu.sync_copy(src_hbm.at[pl.ds(batch0, NBM)], src_v)
        pltpu.sync_copy(dst_hbm.at[pl.ds(batch0, NBM)], dst_v)
        def body(j, _):
            pltpu.async_copy(y_hbm.at[src_v.at[j]], rows_v, sem).wait()
            pltpu.sync_copy(rows_v, acc_sh.at[dst_v.at[j]], add=True)
            return 0
        lax.fori_loop(0, cnt, body, 0)
        plsc.subcore_barrier()

        pltpu.sync_copy(acc_sh.at[pl.ds(base, STRIPE)],
                        out_hbm.at[c].at[pl.ds(base, STRIPE)])
    return k(y_pad, src_f, dst_f)


def _dis_block(dega, degb, pid):
    deg = (jnp.sum(dega, axis=1, keepdims=True)
           + jnp.sum(degb, axis=1, keepdims=True)) * (1.0 / 16.0)
    row = pid * ROW_BLK + lax.broadcasted_iota(jnp.int32, (ROW_BLK, 1), 0)
    deg = deg + jnp.where(row < N_NODES, 1.0, 0.0)
    return jnp.where(deg > 0, lax.rsqrt(deg), 0.0)


def _tc_first(x_pad, W1, dega, degb):
    def body(x_ref, w_ref, da_ref, db_ref, y_ref):
        dis = _dis_block(da_ref[...], db_ref[...], pl.program_id(0))
        y_ref[...] = jnp.dot(x_ref[...], w_ref[...],
                             preferred_element_type=jnp.float32) * dis
    grid = (N_PAD // ROW_BLK,)
    return pl.pallas_call(
        body,
        grid=grid,
        in_specs=[
            pl.BlockSpec((ROW_BLK, D), lambda i: (i, 0)),
            pl.BlockSpec((D, D), lambda i: (0, 0)),
            pl.BlockSpec((ROW_BLK, 16), lambda i: (i, 0)),
            pl.BlockSpec((ROW_BLK, 16), lambda i: (i, 0)),
        ],
        out_specs=pl.BlockSpec((ROW_BLK, D), lambda i: (i, 0)),
        out_shape=jax.ShapeDtypeStruct((N_PAD, D), jnp.float32),
    )(x_pad, W1, dega, degb)


def _tc_mid(s1a, s1b, y1, b1, W2, dega, degb):
    def body(sa_ref, sb_ref, y_ref, b_ref, w_ref, da_ref, db_ref, o_ref):
        dis = _dis_block(da_ref[...], db_ref[...], pl.program_id(0))
        h = dis * (sa_ref[...] + sb_ref[...] + y_ref[...]) + b_ref[...]
        h = jnp.maximum(h, 0.0)
        o_ref[...] = jnp.dot(h, w_ref[...],
                             preferred_element_type=jnp.float32) * dis
    grid = (N_PAD // ROW_BLK,)
    return pl.pallas_call(
        body,
        grid=grid,
        in_specs=[
            pl.BlockSpec((ROW_BLK, D), lambda i: (i, 0)),
            pl.BlockSpec((ROW_BLK, D), lambda i: (i, 0)),
            pl.BlockSpec((ROW_BLK, D), lambda i: (i, 0)),
            pl.BlockSpec((1, D), lambda i: (0, 0)),
            pl.BlockSpec((D, D), lambda i: (0, 0)),
            pl.BlockSpec((ROW_BLK, 16), lambda i: (i, 0)),
            pl.BlockSpec((ROW_BLK, 16), lambda i: (i, 0)),
        ],
        out_specs=pl.BlockSpec((ROW_BLK, D), lambda i: (i, 0)),
        out_shape=jax.ShapeDtypeStruct((N_PAD, D), jnp.float32),
    )(s1a, s1b, y1, b1, W2, dega, degb)


def _tc_last(s2a, s2b, y2, b2, dega, degb):
    def body(sa_ref, sb_ref, y_ref, b_ref, da_ref, db_ref, o_ref):
        dis = _dis_block(da_ref[...], db_ref[...], pl.program_id(0))
        o_ref[...] = dis * (sa_ref[...] + sb_ref[...] + y_ref[...]) + b_ref[...]
    grid = (N_PAD // ROW_BLK,)
    return pl.pallas_call(
        body,
        grid=grid,
        in_specs=[
            pl.BlockSpec((ROW_BLK, D), lambda i: (i, 0)),
            pl.BlockSpec((ROW_BLK, D), lambda i: (i, 0)),
            pl.BlockSpec((ROW_BLK, D), lambda i: (i, 0)),
            pl.BlockSpec((1, D), lambda i: (0, 0)),
            pl.BlockSpec((ROW_BLK, 16), lambda i: (i, 0)),
            pl.BlockSpec((ROW_BLK, 16), lambda i: (i, 0)),
        ],
        out_specs=pl.BlockSpec((ROW_BLK, D), lambda i: (i, 0)),
        out_shape=jax.ShapeDtypeStruct((N_PAD, D), jnp.float32),
    )(s2a, s2b, y2, b2, dega, degb)


def kernel(x, edge_index, W1, b1, W2, b2):
    x_pad = jnp.pad(x, ((0, N_PAD - N_NODES), (0, 0)))
    npad = E_PAD - edge_index.shape[1]
    src_f = jnp.concatenate(
        [edge_index[0], jnp.full((npad,), N_NODES, jnp.int32)]).reshape(NBT, B)
    dst_f = jnp.concatenate(
        [edge_index[1], jnp.full((npad,), N_NODES, jnp.int32)]).reshape(NBT, B)
    dst_t = dst_f.reshape(NW, NB, B)
    b1r = b1.reshape(1, D)
    b2r = b2.reshape(1, D)

    deg = _sc_degree(dst_t)
    dega = jnp.tile(deg[0].reshape(HB * 128, 1)[:N_PAD], (1, 16))
    degb = jnp.tile(deg[1].reshape(HB * 128, 1)[:N_PAD], (1, 16))

    y1 = _tc_first(x_pad, W1, dega, degb)
    s1 = _sc_scatter(y1, src_f, dst_f)
    y2 = _tc_mid(s1[0], s1[1], y1, b1r, W2, dega, degb)
    s2 = _sc_scatter(y2, src_f, dst_f)
    out = _tc_last(s2[0], s2[1], y2, b2r, dega, degb)
    return out[:N_NODES]


# speedup vs baseline: 1.2774x; 1.2774x over previous
import functools

import jax
import jax.numpy as jnp
from jax import lax
from jax.experimental import pallas as pl
from jax.experimental.pallas import tpu as pltpu
from jax.experimental.pallas import tpu_sc as plsc

N_NODES = 10000
N_PAD = 10240
D = 128
NC, NS = 2, 16
NW = NC * NS
B = 128
NB = 80
NBT = NW * NB
NB0, NB1 = 120, 40
NBM = max(NB0, NB1)
E_PAD = NBT * B
STRIPE = N_PAD // NS
ROW_BLK = 128
ZB = 8


def _sc_mesh():
    return plsc.VectorSubcoreMesh(
        core_axis_name="c", subcore_axis_name="s",
        num_cores=NC, num_subcores=NS)


HB = 128


def _sc_degree(dst_t):
    @functools.partial(
        pl.kernel,
        out_type=jax.ShapeDtypeStruct((NC, HB, 128), jnp.float32),
        mesh=_sc_mesh(),
        compiler_params=pltpu.CompilerParams(needs_layout_passes=False),
        scratch_types=[
            pltpu.VMEM((NB, B), jnp.int32),
            pltpu.VMEM((HB, 128), jnp.float32),
            pltpu.VMEM((1, 128), jnp.int32),
            pltpu.VMEM_SHARED((HB, 128), jnp.float32),
        ],
    )
    def k(dst_hbm, out_hbm, dst_v, hist_v, iden_v, acc_sh):
        c = lax.axis_index("c")
        s = lax.axis_index("s")
        wid = c * NS + s

        def zero_row(i, _):
            def zero_lane(jj, _):
                hist_v[i, pl.ds(jj * 16, 16)] = jnp.zeros((16,), jnp.float32)
                return 0
            return lax.fori_loop(0, 128 // 16, zero_lane, 0)
        lax.fori_loop(0, HB, zero_row, 0)
        for kk in range(128 // 16):
            iden_v[0, pl.ds(kk * 16, 16)] = lax.iota(jnp.int32, 16) + 16 * kk
        pltpu.sync_copy(hist_v.at[pl.ds(s * (HB // NS), HB // NS)],
                        acc_sh.at[pl.ds(s * (HB // NS), HB // NS)])
        plsc.subcore_barrier()

        pltpu.sync_copy(dst_hbm.at[wid], dst_v)
        ones16 = jnp.ones((16,), jnp.float32)
        def hrow(jj, _):
            def hlane(kk, _):
                v = dst_v[jj, pl.ds(kk * 16, 16)]
                plsc.addupdate_scatter(
                    hist_v, [jnp.right_shift(v, 7), jnp.bitwise_and(v, 127)],
                    ones16)
                return 0
            return lax.fori_loop(0, B // 16, hlane, 0)
        lax.fori_loop(0, NB, hrow, 0)

        pltpu.sync_copy(hist_v, acc_sh.at[iden_v.at[0]], add=True)
        plsc.subcore_barrier()

        base = s * (HB // NS)
        pltpu.sync_copy(acc_sh.at[pl.ds(base, HB // NS)],
                        out_hbm.at[c].at[pl.ds(base, HB // NS)])
    return k(dst_t)


def _sc_scatter(y_pad, src_f, dst_f):
    @functools.partial(
        pl.kernel,
        out_type=jax.ShapeDtypeStruct((NC, N_PAD, D), jnp.float32),
        mesh=_sc_mesh(),
        scratch_types=[
            pltpu.VMEM((NBM, B), jnp.int32),
            pltpu.VMEM((NBM, B), jnp.int32),
            pltpu.VMEM((B, D), jnp.float32),
            pltpu.VMEM((ZB, D), jnp.float32),
            pltpu.VMEM_SHARED((N_PAD, D), jnp.float32),
            pltpu.SemaphoreType.DMA,
        ],
    )
    def k(y_hbm, src_hbm, dst_hbm, out_hbm,
          src_v, dst_v, rows_v, zero_v, acc_sh, sem):
        c = lax.axis_index("c")
        s = lax.axis_index("s")

        def zero_row(i, _):
            def zero_lane(jj, _):
                zero_v[i, pl.ds(jj * 16, 16)] = jnp.zeros((16,), jnp.float32)
                return 0
            return lax.fori_loop(0, D // 16, zero_lane, 0)
        lax.fori_loop(0, ZB, zero_row, 0)

        base = s * STRIPE
        def zero_stripe(kk, _):
            pltpu.sync_copy(zero_v, acc_sh.at[pl.ds(base + kk * ZB, ZB)])
            return 0
        lax.fori_loop(0, STRIPE // ZB, zero_stripe, 0)
        plsc.subcore_barrier()

        batch0 = jnp.where(c == 0, s * NB0, NS * NB0 + s * NB1)
        cnt = jnp.where(c == 0, NB0, NB1)
        pltpu.sync_copy(src_hbm.at[pl.ds(batch0, NBM)], src_v)
        pltpu.sync_copy(dst_hbm.at[pl.ds(batch0, NBM)], dst_v)
        def body(j, _):
            pltpu.async_copy(y_hbm.at[src_v.at[j]], rows_v, sem).wait()
            pltpu.sync_copy(rows_v, acc_sh.at[dst_v.at[j]], add=True)
            return 0
        lax.fori_loop(0, cnt, body, 0)
        plsc.subcore_barrier()

        pltpu.sync_copy(acc_sh.at[pl.ds(base, STRIPE)],
                        out_hbm.at[c].at[pl.ds(base, STRIPE)])
    return k(y_pad, src_f, dst_f)


def _dis_block(dega, degb, pid):
    deg = (jnp.sum(dega, axis=1, keepdims=True)
           + jnp.sum(degb, axis=1, keepdims=True)) * (1.0 / 16.0)
    row = pid * ROW_BLK + lax.broadcasted_iota(jnp.int32, (ROW_BLK, 1), 0)
    deg = deg + jnp.where(row < N_NODES, 1.0, 0.0)
    return jnp.where(deg > 0, lax.rsqrt(deg), 0.0)


def _tc_first(x_pad, W1, dega, degb):
    def body(x_ref, w_ref, da_ref, db_ref, y_ref):
        dis = _dis_block(da_ref[...], db_ref[...], pl.program_id(0))
        y_ref[...] = jnp.dot(x_ref[...], w_ref[...],
                             preferred_element_type=jnp.float32) * dis
    grid = (N_PAD // ROW_BLK,)
    return pl.pallas_call(
        body,
        grid=grid,
        in_specs=[
            pl.BlockSpec((ROW_BLK, D), lambda i: (i, 0)),
            pl.BlockSpec((D, D), lambda i: (0, 0)),
            pl.BlockSpec((ROW_BLK, 16), lambda i: (i, 0)),
            pl.BlockSpec((ROW_BLK, 16), lambda i: (i, 0)),
        ],
        out_specs=pl.BlockSpec((ROW_BLK, D), lambda i: (i, 0)),
        out_shape=jax.ShapeDtypeStruct((N_PAD, D), jnp.float32),
    )(x_pad, W1, dega, degb)


def _tc_mid(s1a, s1b, y1, b1, W2, dega, degb):
    def body(sa_ref, sb_ref, y_ref, b_ref, w_ref, da_ref, db_ref, o_ref):
        dis = _dis_block(da_ref[...], db_ref[...], pl.program_id(0))
        h = dis * (sa_ref[...] + sb_ref[...] + y_ref[...]) + b_ref[...]
        h = jnp.maximum(h, 0.0)
        o_ref[...] = jnp.dot(h, w_ref[...],
                             preferred_element_type=jnp.float32) * dis
    grid = (N_PAD // ROW_BLK,)
    return pl.pallas_call(
        body,
        grid=grid,
        in_specs=[
            pl.BlockSpec((ROW_BLK, D), lambda i: (i, 0)),
            pl.BlockSpec((ROW_BLK, D), lambda i: (i, 0)),
            pl.BlockSpec((ROW_BLK, D), lambda i: (i, 0)),
            pl.BlockSpec((1, D), lambda i: (0, 0)),
            pl.BlockSpec((D, D), lambda i: (0, 0)),
            pl.BlockSpec((ROW_BLK, 16), lambda i: (i, 0)),
            pl.BlockSpec((ROW_BLK, 16), lambda i: (i, 0)),
        ],
        out_specs=pl.BlockSpec((ROW_BLK, D), lambda i: (i, 0)),
        out_shape=jax.ShapeDtypeStruct((N_PAD, D), jnp.float32),
    )(s1a, s1b, y1, b1, W2, dega, degb)


def _tc_last(s2a, s2b, y2, b2, dega, degb):
    def body(sa_ref, sb_ref, y_ref, b_ref, da_ref, db_ref, o_ref):
        dis = _dis_block(da_ref[...], db_ref[...], pl.program_id(0))
        o_ref[...] = dis * (sa_ref[...] + sb_ref[...] + y_ref[...]) + b_ref[...]
    grid = (N_PAD // ROW_BLK,)
    return pl.pallas_call(
        body,
        grid=grid,
        in_specs=[
            pl.BlockSpec((ROW_BLK, D), lambda i: (i, 0)),
            pl.BlockSpec((ROW_BLK, D), lambda i: (i, 0)),
            pl.BlockSpec((ROW_BLK, D), lambda i: (i, 0)),
            pl.BlockSpec((1, D), lambda i: (0, 0)),
            pl.BlockSpec((ROW_BLK, 16), lambda i: (i, 0)),
            pl.BlockSpec((ROW_BLK, 16), lambda i: (i, 0)),
        ],
        out_specs=pl.BlockSpec((ROW_BLK, D), lambda i: (i, 0)),
        out_shape=jax.ShapeDtypeStruct((N_PAD, D), jnp.float32),
    )(s2a, s2b, y2, b2, dega, degb)


def kernel(x, edge_index, W1, b1, W2, b2):
    x_pad = jnp.pad(x, ((0, N_PAD - N_NODES), (0, 0)))
    npad = E_PAD - edge_index.shape[1]
    src_f = jnp.concatenate(
        [edge_index[0], jnp.full((npad,), N_NODES, jnp.int32)]).reshape(NBT, B)
    dst_f = jnp.concatenate(
        [edge_index[1], jnp.full((npad,), N_NODES, jnp.int32)]).reshape(NBT, B)
    dst_t = dst_f.reshape(NW, NB, B)
    b1r = b1.reshape(1, D)
    b2r = b2.reshape(1, D)

    deg = _sc_degree(dst_t)
    dega = jnp.tile(deg[0].reshape(HB * 128, 1)[:N_PAD], (1, 16))
    degb = jnp.tile(deg[1].reshape(HB * 128, 1)[:N_PAD], (1, 16))

    y1 = _tc_first(x_pad, W1, dega, degb)
    s1 = _sc_scatter(y1, src_f, dst_f)
    y2 = _tc_mid(s1[0], s1[1], y1, b1r, W2, dega, degb)
    s2 = _sc_scatter(y2, src_f, dst_f)
    out = _tc_last(s2[0], s2[1], y2, b2r, dega, degb)
    return out[:N_NODES]
